# Initial kernel scaffold; baseline (speedup 1.0000x reference)
import jax
import jax.numpy as jnp
from jax.experimental import pallas as pl


def kernel(pos_v, pos_u, neg_u, v_weight, u_weight):
    raise NotImplementedError("write your pallas kernel here")


# speedup vs baseline: 2.5182x; 2.5182x over previous
import functools

import jax
import jax.numpy as jnp
from jax import lax
from jax.experimental import pallas as pl
from jax.experimental.pallas import tpu as pltpu
from jax.experimental.pallas import tpu_sc as plsc

V = 100000
D = 128
B = 16384
K = 5
L = 16
NC = 2
NS = 16
NW = NC * NS
BPW = B // NW
C = 64
G = BPW // C
NV = D // L


def _sc_scores(pos_v, pos_u, neg_u, v_weight, u_weight):
    mesh = plsc.VectorSubcoreMesh(core_axis_name="c", subcore_axis_name="s")

    @functools.partial(
        pl.kernel,
        mesh=mesh,
        out_type=[
            jax.ShapeDtypeStruct((B, L), jnp.float32),
            jax.ShapeDtypeStruct((B * K, L), jnp.float32),
        ],
        scratch_types=[
            pltpu.VMEM((G, C), jnp.int32),
            pltpu.VMEM((G, C), jnp.int32),
            pltpu.VMEM((G * K, C), jnp.int32),
            pltpu.VMEM((C, D), jnp.float32),
            pltpu.VMEM((C, D), jnp.float32),
            pltpu.VMEM((C * K, D), jnp.float32),
            pltpu.VMEM((C, L), jnp.float32),
            pltpu.VMEM((C * K, L), jnp.float32),
            pltpu.SemaphoreType.DMA,
        ],
    )
    def k(pv_hbm, pu_hbm, ng_hbm, vw_hbm, uw_hbm, pos_out, neg_out,
          pv_idx, pu_idx, ng_idx, v_rows, u_rows, n_rows, p_sc, n_sc, sem):
        wid = lax.axis_index("s") * NC + lax.axis_index("c")
        pltpu.sync_copy(pv_hbm.at[wid], pv_idx)
        pltpu.sync_copy(pu_hbm.at[wid], pu_idx)
        pltpu.sync_copy(ng_hbm.at[wid], ng_idx)

        def group(g, carry):
            cps = [
                pltpu.async_copy(vw_hbm.at[pv_idx.at[g]], v_rows, sem),
                pltpu.async_copy(uw_hbm.at[pu_idx.at[g]], u_rows, sem),
            ]
            for kk in range(K):
                cps.append(pltpu.async_copy(
                    uw_hbm.at[ng_idx.at[g * K + kk]],
                    n_rows.at[pl.ds(kk * C, C)], sem))
            for cp in cps:
                cp.wait()

            def elem(i, carry2):
                vv = [v_rows[i, pl.ds(L * j, L)] for j in range(NV)]
                acc = vv[0] * u_rows[i, pl.ds(0, L)]
                for j in range(1, NV):
                    acc = acc + vv[j] * u_rows[i, pl.ds(L * j, L)]
                p_sc[i, :] = acc
                for kk in range(K):
                    r = i * K + kk
                    acc2 = vv[0] * n_rows[r, pl.ds(0, L)]
                    for j in range(1, NV):
                        acc2 = acc2 + vv[j] * n_rows[r, pl.ds(L * j, L)]
                    n_sc[r, :] = acc2
                return carry2

            lax.fori_loop(0, C, elem, 0)
            base = wid * BPW + g * C
            pltpu.sync_copy(p_sc, pos_out.at[pl.ds(base, C)])
            nbase = (wid * BPW + g * C) * K
            pltpu.sync_copy(n_sc, neg_out.at[pl.ds(nbase, C * K)])
            return carry

        lax.fori_loop(0, G, group, 0)

    return k(pos_v, pos_u, neg_u, v_weight, u_weight)


def _tc_reduce_body(pos_ref, neg_ref, out_ref):
    p = jnp.sum(pos_ref[...], axis=-1)
    n = jnp.sum(neg_ref[...], axis=-1)
    tot = jnp.sum(jax.nn.log_sigmoid(p)) + jnp.sum(jax.nn.log_sigmoid(-n))
    out_ref[0, 0] = -tot


def _tc_reduce(pos_part, neg_part):
    return pl.pallas_call(
        _tc_reduce_body,
        out_shape=jax.ShapeDtypeStruct((1, 1), jnp.float32),
        out_specs=pl.BlockSpec(memory_space=pltpu.SMEM),
    )(pos_part, neg_part)


def kernel(pos_v, pos_u, neg_u, v_weight, u_weight):
    pos_v = pos_v.astype(jnp.int32).reshape(NW, G, C)
    pos_u = pos_u.astype(jnp.int32).reshape(NW, G, C)
    neg_u = neg_u.astype(jnp.int32).reshape(NW, G * K, C)
    pos_part, neg_part = _sc_scores(pos_v, pos_u, neg_u, v_weight, u_weight)
    out = _tc_reduce(pos_part, neg_part)
    return out[0, 0]


# speedup vs baseline: 4.5806x; 1.8190x over previous
import functools

import jax
import jax.numpy as jnp
from jax import lax
from jax.experimental import pallas as pl
from jax.experimental.pallas import tpu as pltpu
from jax.experimental.pallas import tpu_sc as plsc

V = 100000
D = 128
B = 16384
K = 5
L = 16
NC = 2
NS = 16
NW = NC * NS
BPW = B // NW
C = 64
G = BPW // C
NV = D // L


def _sc_scores(pos_v, pos_u, neg_u, v_weight, u_weight):
    mesh = plsc.VectorSubcoreMesh(core_axis_name="c", subcore_axis_name="s")

    @functools.partial(
        pl.kernel,
        mesh=mesh,
        out_type=[
            jax.ShapeDtypeStruct((B * L,), jnp.float32),
            jax.ShapeDtypeStruct((B * K * L,), jnp.float32),
        ],
        scratch_types=[
            pltpu.VMEM((BPW,), jnp.int32),
            pltpu.VMEM((BPW,), jnp.int32),
            pltpu.VMEM((BPW * K,), jnp.int32),
            pltpu.VMEM((C, D), jnp.float32),
            pltpu.VMEM((C, D), jnp.float32),
            pltpu.VMEM((C, D), jnp.float32),
            pltpu.VMEM((C, D), jnp.float32),
            pltpu.VMEM((C * K, D), jnp.float32),
            pltpu.VMEM((C * K, D), jnp.float32),
            pltpu.VMEM((C * L,), jnp.float32),
            pltpu.VMEM((C * K * L,), jnp.float32),
            pltpu.SemaphoreType.DMA,
            pltpu.SemaphoreType.DMA,
        ],
    )
    def k(pv_hbm, pu_hbm, ng_hbm, vw_hbm, uw_hbm, pos_out, neg_out,
          pv_idx, pu_idx, ng_idx, v_a, v_b, u_a, u_b, n_a, n_b,
          p_sc, n_sc, sem_a, sem_b):
        wid = lax.axis_index("s") * NC + lax.axis_index("c")
        pltpu.sync_copy(pv_hbm.at[pl.ds(wid * BPW, BPW)], pv_idx)
        pltpu.sync_copy(pu_hbm.at[pl.ds(wid * BPW, BPW)], pu_idx)
        pltpu.sync_copy(ng_hbm.at[pl.ds(wid * BPW * K, BPW * K)], ng_idx)

        bufs = [(v_a, u_a, n_a, sem_a), (v_b, u_b, n_b, sem_b)]

        def fire(g):
            v_r, u_r, n_r, sem = bufs[g % 2]
            cps = [
                pltpu.async_copy(
                    vw_hbm.at[pv_idx.at[pl.ds(g * C, C)]], v_r, sem),
                pltpu.async_copy(
                    uw_hbm.at[pu_idx.at[pl.ds(g * C, C)]], u_r, sem),
            ]
            for kk in range(K):
                cps.append(pltpu.async_copy(
                    uw_hbm.at[ng_idx.at[pl.ds(g * C * K + kk * C, C)]],
                    n_r.at[pl.ds(kk * C, C)], sem))
            return cps

        cps = fire(0)
        for g in range(G):
            nxt = fire(g + 1) if g + 1 < G else []
            for cp in cps:
                cp.wait()
            v_r, u_r, n_r, _ = bufs[g % 2]

            def elem(i, carry, v_r=v_r, u_r=u_r, n_r=n_r):
                vv = [v_r[i, pl.ds(L * j, L)] for j in range(NV)]
                acc = vv[0] * u_r[i, pl.ds(0, L)]
                for j in range(1, NV):
                    acc = acc + vv[j] * u_r[i, pl.ds(L * j, L)]
                p_sc[pl.ds(i * L, L)] = acc
                for kk in range(K):
                    r = i * K + kk
                    acc2 = vv[0] * n_r[r, pl.ds(0, L)]
                    for j in range(1, NV):
                        acc2 = acc2 + vv[j] * n_r[r, pl.ds(L * j, L)]
                    n_sc[pl.ds(r * L, L)] = acc2
                return carry

            lax.fori_loop(0, C, elem, 0)
            base = (wid * BPW + g * C) * L
            pltpu.sync_copy(p_sc, pos_out.at[pl.ds(base, C * L)])
            nbase = (wid * BPW + g * C) * K * L
            pltpu.sync_copy(n_sc, neg_out.at[pl.ds(nbase, C * K * L)])
            cps = nxt

    return k(pos_v, pos_u, neg_u, v_weight, u_weight)


def _fold16(x):
    for sh in (1, 2, 4, 8):
        x = x + jnp.roll(x, -sh, axis=1)
    return x


def _tc_reduce_body(pos_ref, neg_ref, out_ref):
    p = _fold16(pos_ref[...])
    n = _fold16(neg_ref[...])
    pm = lax.broadcasted_iota(jnp.int32, p.shape, 1) % L == 0
    nm = lax.broadcasted_iota(jnp.int32, n.shape, 1) % L == 0
    tot = jnp.sum(jnp.where(pm, jax.nn.log_sigmoid(p), 0.0))
    tot = tot + jnp.sum(jnp.where(nm, jax.nn.log_sigmoid(-n), 0.0))
    out_ref[0, 0] = -tot


def _tc_reduce(pos_p, neg_p):
    return pl.pallas_call(
        _tc_reduce_body,
        out_shape=jax.ShapeDtypeStruct((1, 1), jnp.float32),
        out_specs=pl.BlockSpec(memory_space=pltpu.SMEM),
    )(pos_p.reshape(B * L // D, D), neg_p.reshape(B * K * L // D, D))


def kernel(pos_v, pos_u, neg_u, v_weight, u_weight):
    pos_v = pos_v.astype(jnp.int32)
    pos_u = pos_u.astype(jnp.int32)
    neg_u = neg_u.astype(jnp.int32).reshape(B * K)
    pos_p, neg_p = _sc_scores(pos_v, pos_u, neg_u, v_weight, u_weight)
    out = _tc_reduce(pos_p, neg_p)
    return out[0, 0]


# speedup vs baseline: 4.6036x; 1.0050x over previous
import functools

import jax
import jax.numpy as jnp
from jax import lax
from jax.experimental import pallas as pl
from jax.experimental.pallas import tpu as pltpu
from jax.experimental.pallas import tpu_sc as plsc

V = 100000
D = 128
B = 16384
K = 5
L = 16
NC = 2
NS = 16
NW = NC * NS
BPW = B // NW
C = 32
G = BPW // C
NV = D // L


def _sc_scores(pos_v, pos_u, neg_u, v_weight, u_weight):
    mesh = plsc.VectorSubcoreMesh(core_axis_name="c", subcore_axis_name="s")

    @functools.partial(
        pl.kernel,
        mesh=mesh,
        out_type=[
            jax.ShapeDtypeStruct((B * L,), jnp.float32),
            jax.ShapeDtypeStruct((B * K * L,), jnp.float32),
        ],
        scratch_types=[
            pltpu.VMEM((BPW,), jnp.int32),
            pltpu.VMEM((BPW,), jnp.int32),
            pltpu.VMEM((BPW * K,), jnp.int32),
            pltpu.VMEM((C, D), jnp.float32),
            pltpu.VMEM((C, D), jnp.float32),
            pltpu.VMEM((C, D), jnp.float32),
            pltpu.VMEM((C, D), jnp.float32),
            pltpu.VMEM((C * K, D), jnp.float32),
            pltpu.VMEM((C * K, D), jnp.float32),
            pltpu.VMEM((BPW * L,), jnp.float32),
            pltpu.VMEM((BPW * K * L,), jnp.float32),
            pltpu.SemaphoreType.DMA,
            pltpu.SemaphoreType.DMA,
        ],
    )
    def k(pv_hbm, pu_hbm, ng_hbm, vw_hbm, uw_hbm, pos_out, neg_out,
          pv_idx, pu_idx, ng_idx, v_a, v_b, u_a, u_b, n_a, n_b,
          p_all, n_all, sem_a, sem_b):
        wid = lax.axis_index("s") * NC + lax.axis_index("c")
        pltpu.sync_copy(pv_hbm.at[pl.ds(wid * BPW, BPW)], pv_idx)
        pltpu.sync_copy(pu_hbm.at[pl.ds(wid * BPW, BPW)], pu_idx)
        pltpu.sync_copy(ng_hbm.at[pl.ds(wid * BPW * K, BPW * K)], ng_idx)

        bufs = [(v_a, u_a, n_a, sem_a), (v_b, u_b, n_b, sem_b)]

        def fire(g):
            v_r, u_r, n_r, sem = bufs[g % 2]
            cps = [
                pltpu.async_copy(
                    vw_hbm.at[pv_idx.at[pl.ds(g * C, C)]], v_r, sem),
                pltpu.async_copy(
                    uw_hbm.at[pu_idx.at[pl.ds(g * C, C)]], u_r, sem),
            ]
            for kk in range(K):
                cps.append(pltpu.async_copy(
                    uw_hbm.at[ng_idx.at[pl.ds(g * C * K + kk * C, C)]],
                    n_r.at[pl.ds(kk * C, C)], sem))
            return cps

        cps = fire(0)
        for g in range(G):
            nxt = fire(g + 1) if g + 1 < G else []
            for cp in cps:
                cp.wait()
            v_r, u_r, n_r, _ = bufs[g % 2]

            def elem(i, carry, v_r=v_r, u_r=u_r, n_r=n_r, g=g):
                vv = [v_r[i, pl.ds(L * j, L)] for j in range(NV)]
                acc = vv[0] * u_r[i, pl.ds(0, L)]
                for j in range(1, NV):
                    acc = acc + vv[j] * u_r[i, pl.ds(L * j, L)]
                p_all[pl.ds(g * C * L + i * L, L)] = acc
                for kk in range(K):
                    r = i * K + kk
                    acc2 = vv[0] * n_r[r, pl.ds(0, L)]
                    for j in range(1, NV):
                        acc2 = acc2 + vv[j] * n_r[r, pl.ds(L * j, L)]
                    n_all[pl.ds(g * C * K * L + r * L, L)] = acc2
                return carry

            lax.fori_loop(0, C, elem, 0)
            cps = nxt

        pltpu.sync_copy(p_all, pos_out.at[pl.ds(wid * BPW * L, BPW * L)])
        pltpu.sync_copy(n_all, neg_out.at[pl.ds(wid * BPW * K * L, BPW * K * L)])

    return k(pos_v, pos_u, neg_u, v_weight, u_weight)


def _fold16(x):
    r = lax.broadcasted_iota(jnp.int32, (D, D), 0)
    c = lax.broadcasted_iota(jnp.int32, (D, D), 1)
    m = jnp.where(r // L == c // L, 1.0, 0.0).astype(jnp.float32)
    return jax.lax.dot(x, m, precision=jax.lax.Precision.HIGHEST)


def _tc_reduce_body(pos_ref, neg_ref, out_ref):
    p = _fold16(pos_ref[...])
    n = _fold16(neg_ref[...])
    pm = lax.broadcasted_iota(jnp.int32, p.shape, 1) % L == 0
    nm = lax.broadcasted_iota(jnp.int32, n.shape, 1) % L == 0
    tot = jnp.sum(jnp.where(pm, jax.nn.log_sigmoid(p), 0.0))
    tot = tot + jnp.sum(jnp.where(nm, jax.nn.log_sigmoid(-n), 0.0))
    out_ref[0, 0] = -tot


def _tc_reduce(pos_p, neg_p):
    return pl.pallas_call(
        _tc_reduce_body,
        out_shape=jax.ShapeDtypeStruct((1, 1), jnp.float32),
        out_specs=pl.BlockSpec(memory_space=pltpu.SMEM),
    )(pos_p.reshape(B * L // D, D), neg_p.reshape(B * K * L // D, D))


def kernel(pos_v, pos_u, neg_u, v_weight, u_weight):
    pos_v = pos_v.astype(jnp.int32)
    pos_u = pos_u.astype(jnp.int32)
    neg_u = neg_u.astype(jnp.int32).reshape(B * K)
    pos_p, neg_p = _sc_scores(pos_v, pos_u, neg_u, v_weight, u_weight)
    out = _tc_reduce(pos_p, neg_p)
    return out[0, 0]


# speedup vs baseline: 5.3850x; 1.1697x over previous
import functools

import jax
import jax.numpy as jnp
from jax import lax
from jax.experimental import pallas as pl
from jax.experimental.pallas import tpu as pltpu
from jax.experimental.pallas import tpu_sc as plsc

V = 100000
D = 128
B = 16384
K = 5
L = 16
NC = 2
NS = 16
NW = NC * NS
BPW = B // NW
C = 32
G = BPW // C
NV = D // L
TCG = 8


def _sc_scores(pos_v, pos_u, neg_u, v_weight, u_weight):
    mesh = plsc.VectorSubcoreMesh(core_axis_name="c", subcore_axis_name="s")

    @functools.partial(
        pl.kernel,
        mesh=mesh,
        out_type=[
            jax.ShapeDtypeStruct((B * L,), jnp.float32),
            jax.ShapeDtypeStruct((B * K * L,), jnp.float32),
        ],
        scratch_types=[
            pltpu.VMEM((BPW,), jnp.int32),
            pltpu.VMEM((BPW,), jnp.int32),
            pltpu.VMEM((BPW * K,), jnp.int32),
            pltpu.VMEM((C, D), jnp.float32),
            pltpu.VMEM((C, D), jnp.float32),
            pltpu.VMEM((C, D), jnp.float32),
            pltpu.VMEM((C, D), jnp.float32),
            pltpu.VMEM((C * K, D), jnp.float32),
            pltpu.VMEM((C * K, D), jnp.float32),
            pltpu.VMEM((BPW * L,), jnp.float32),
            pltpu.VMEM((BPW * K * L,), jnp.float32),
            pltpu.SemaphoreType.DMA,
            pltpu.SemaphoreType.DMA,
        ],
    )
    def k(pv_hbm, pu_hbm, ng_hbm, vw_hbm, uw_hbm, pos_out, neg_out,
          pv_idx, pu_idx, ng_idx, v_a, v_b, u_a, u_b, n_a, n_b,
          p_all, n_all, sem_a, sem_b):
        wid = lax.axis_index("s") * NC + lax.axis_index("c")
        pltpu.sync_copy(pv_hbm.at[pl.ds(wid * BPW, BPW)], pv_idx)
        pltpu.sync_copy(pu_hbm.at[pl.ds(wid * BPW, BPW)], pu_idx)
        pltpu.sync_copy(ng_hbm.at[pl.ds(wid * BPW * K, BPW * K)], ng_idx)

        bufs = [(v_a, u_a, n_a, sem_a), (v_b, u_b, n_b, sem_b)]

        def fire(g):
            v_r, u_r, n_r, sem = bufs[g % 2]
            cps = [
                pltpu.async_copy(
                    vw_hbm.at[pv_idx.at[pl.ds(g * C, C)]], v_r, sem),
                pltpu.async_copy(
                    uw_hbm.at[pu_idx.at[pl.ds(g * C, C)]], u_r, sem),
            ]
            for kk in range(K):
                cps.append(pltpu.async_copy(
                    uw_hbm.at[ng_idx.at[pl.ds(g * C * K + kk * C, C)]],
                    n_r.at[pl.ds(kk * C, C)], sem))
            return cps

        cps = fire(0)
        for g in range(G):
            nxt = fire(g + 1) if g + 1 < G else []
            for cp in cps:
                cp.wait()
            v_r, u_r, n_r, _ = bufs[g % 2]

            @plsc.parallel_loop(0, C, unroll=2)
            def elem(i, v_r=v_r, u_r=u_r, n_r=n_r, g=g):
                vv = [v_r[i, pl.ds(L * j, L)] for j in range(NV)]
                acc = vv[0] * u_r[i, pl.ds(0, L)]
                for j in range(1, NV):
                    acc = acc + vv[j] * u_r[i, pl.ds(L * j, L)]
                p_all[pl.ds(g * C * L + i * L, L)] = acc
                for kk in range(K):
                    r = i * K + kk
                    acc2 = vv[0] * n_r[r, pl.ds(0, L)]
                    for j in range(1, NV):
                        acc2 = acc2 + vv[j] * n_r[r, pl.ds(L * j, L)]
                    n_all[pl.ds(g * C * K * L + r * L, L)] = acc2

            cps = nxt

        pltpu.sync_copy(p_all, pos_out.at[pl.ds(wid * BPW * L, BPW * L)])
        pltpu.sync_copy(n_all, neg_out.at[pl.ds(wid * BPW * K * L, BPW * K * L)])

    return k(pos_v, pos_u, neg_u, v_weight, u_weight)


def _block_loss(x, sign):
    r = lax.broadcasted_iota(jnp.int32, (D, D), 0)
    c = lax.broadcasted_iota(jnp.int32, (D, D), 1)
    m = jnp.where(r // L == c // L, 1.0, 0.0).astype(jnp.float32)
    folded = jax.lax.dot(x, m, precision=jax.lax.Precision.HIGHEST)
    keep = lax.broadcasted_iota(jnp.int32, folded.shape, 1) % L == 0
    return jnp.sum(jnp.where(keep, jax.nn.log_sigmoid(sign * folded), 0.0))


def _tc_reduce_body(pos_ref, neg_ref, out_ref):
    i = pl.program_id(0)
    part = _block_loss(pos_ref[...], 1.0) + _block_loss(neg_ref[...], -1.0)

    @pl.when(i == 0)
    def _():
        out_ref[0, 0] = -part

    @pl.when(i > 0)
    def _():
        out_ref[0, 0] = out_ref[0, 0] - part


def _tc_reduce(pos_p, neg_p):
    pr = B * L // D
    nr = B * K * L // D
    return pl.pallas_call(
        _tc_reduce_body,
        grid=(TCG,),
        in_specs=[
            pl.BlockSpec((pr // TCG, D), lambda i: (i, 0)),
            pl.BlockSpec((nr // TCG, D), lambda i: (i, 0)),
        ],
        out_specs=pl.BlockSpec((1, 1), lambda i: (0, 0),
                               memory_space=pltpu.SMEM),
        out_shape=jax.ShapeDtypeStruct((1, 1), jnp.float32),
    )(pos_p.reshape(pr, D), neg_p.reshape(nr, D))


def kernel(pos_v, pos_u, neg_u, v_weight, u_weight):
    pos_v = pos_v.astype(jnp.int32)
    pos_u = pos_u.astype(jnp.int32)
    neg_u = neg_u.astype(jnp.int32).reshape(B * K)
    pos_p, neg_p = _sc_scores(pos_v, pos_u, neg_u, v_weight, u_weight)
    out = _tc_reduce(pos_p, neg_p)
    return out[0, 0]
